# Initial kernel scaffold; baseline (speedup 1.0000x reference)
import jax
import jax.numpy as jnp
from jax.experimental import pallas as pl


def kernel(input_ids, table, W1, b1, W2, b2):
    raise NotImplementedError("write your pallas kernel here")


# speedup vs baseline: 15.6066x; 15.6066x over previous
import jax
import jax.numpy as jnp
from jax import lax
from jax.experimental import pallas as pl
from jax.experimental.pallas import tpu as pltpu
from jax.experimental.pallas import tpu_sc as plsc

_NC, _NS = 2, 16
_NW = _NC * _NS
_LANES = 16


def _sc_pool(ids2, table, B, L, D):
  HALF = L // 2
  BPW = B // _NW
  OUTC = 64
  NOC = BPW // OUTC
  GROUP = 4
  NG = OUTC // GROUP
  ROWS = GROUP * L
  DH = D // _LANES

  mesh = plsc.VectorSubcoreMesh(core_axis_name="c", subcore_axis_name="s",
                                num_cores=_NC, num_subcores=_NS)

  def body(ids_hbm, table_hbm, out_hbm, ids_v, buf0, buf1, out_v, sem0, sem1):
    bufs = (buf0, buf1)
    sems = (sem0, sem1)
    wid = lax.axis_index("c") * _NS + lax.axis_index("s")

    def fire(g, b):
      descs = []
      for r in range(GROUP):
        for h in range(2):
          irow = 2 * (g * GROUP + r) + h
          dst = bufs[b].at[pl.ds(r * L + h * HALF, HALF)]
          descs.append(pltpu.async_copy(
              table_hbm.at[ids_v.at[irow]], dst, sems[b]))
      return descs

    def reduce_group(g, b):
      buf = bufs[b]
      zero = jnp.zeros((_LANES,), jnp.float32)

      def rbody(j, carry):
        cs = list(carry)
        for u in range(2):
          for r in range(GROUP):
            row = r * L + 2 * j + u
            for h in range(DH):
              k = r * DH + h
              cs[k] = cs[k] + buf[row, pl.ds(h * _LANES, _LANES)]
        return tuple(cs)

      acc = lax.fori_loop(0, L // 2, rbody, (zero,) * (GROUP * DH))
      for r in range(GROUP):
        for h in range(DH):
          out_v[g * GROUP + r, pl.ds(h * _LANES, _LANES)] = acc[r * DH + h]

    def oc_body(oc, carry):
      ids_base = wid * (BPW * 2) + oc * (2 * OUTC)
      pltpu.sync_copy(ids_hbm.at[pl.ds(ids_base, 2 * OUTC)], ids_v)
      descs = [None, None]
      descs[0] = fire(0, 0)
      for g in range(NG):
        b = g % 2
        if g + 1 < NG:
          descs[1 - b] = fire(g + 1, 1 - b)
        for dsc in descs[b]:
          dsc.wait()
        reduce_group(g, b)
      pltpu.sync_copy(out_v, out_hbm.at[pl.ds(wid * BPW + oc * OUTC, OUTC)])
      return carry

    lax.fori_loop(0, NOC, oc_body, 0)

  k = pl.kernel(
      body,
      out_type=jax.ShapeDtypeStruct((B, D), jnp.float32),
      mesh=mesh,
      compiler_params=pltpu.CompilerParams(use_tc_tiling_on_sc=False),
      scratch_types=[
          pltpu.VMEM((2 * OUTC, HALF), jnp.int32),
          pltpu.VMEM((ROWS, D), jnp.float32),
          pltpu.VMEM((ROWS, D), jnp.float32),
          pltpu.VMEM((OUTC, D), jnp.float32),
          pltpu.SemaphoreType.DMA,
          pltpu.SemaphoreType.DMA,
      ],
  )
  return k(ids2, table)


def _mlp(sums, input_ids, W1, b1, W2, b2):
  B, L = input_ids.shape
  D = sums.shape[1]
  H = W1.shape[0]
  T = W2.shape[0]
  BLK = 1024

  def body(sums_ref, ids_ref, W1_ref, b1_ref, W2_ref, b2_ref, out_ref):
    ids = ids_ref[...]
    cnt = jnp.sum((ids != 0).astype(jnp.float32), axis=1, keepdims=True)
    mean = sums_ref[...] / jnp.maximum(cnt, 1.0)
    h = lax.dot_general(mean, W1_ref[...], (((1,), (1,)), ((), ())),
                        preferred_element_type=jnp.float32) + b1_ref[...]
    h = jnp.maximum(h, 0.0)
    out_ref[...] = lax.dot_general(h, W2_ref[...], (((1,), (1,)), ((), ())),
                                   preferred_element_type=jnp.float32) + b2_ref[...]

  return pl.pallas_call(
      body,
      grid=(B // BLK,),
      in_specs=[
          pl.BlockSpec((BLK, D), lambda i: (i, 0)),
          pl.BlockSpec((BLK, L), lambda i: (i, 0)),
          pl.BlockSpec((H, D), lambda i: (0, 0)),
          pl.BlockSpec((1, H), lambda i: (0, 0)),
          pl.BlockSpec((T, H), lambda i: (0, 0)),
          pl.BlockSpec((1, T), lambda i: (0, 0)),
      ],
      out_specs=pl.BlockSpec((BLK, T), lambda i: (i, 0)),
      out_shape=jax.ShapeDtypeStruct((B, T), jnp.float32),
  )(sums, input_ids, W1, b1.reshape(1, H), W2, b2.reshape(1, T))


def kernel(input_ids, table, W1, b1, W2, b2):
  B, L = input_ids.shape
  D = table.shape[1]
  ids2 = input_ids.reshape(B * 2, L // 2)
  sums = _sc_pool(ids2, table, B, L, D)
  return _mlp(sums, input_ids, W1, b1, W2, b2)


# speedup vs baseline: 15.8245x; 1.0140x over previous
import jax
import jax.numpy as jnp
from jax import lax
from jax.experimental import pallas as pl
from jax.experimental.pallas import tpu as pltpu
from jax.experimental.pallas import tpu_sc as plsc

_NC, _NS = 2, 16
_NW = _NC * _NS
_LANES = 16


def _sc_pool(ids, table, B, L, D):
  SPLITS = ((0, 104), (104, 96))
  BPW = B // _NW
  OUTC = 64
  NOC = BPW // OUTC
  GROUP = 4
  NG = OUTC // GROUP
  ROWS = GROUP * L
  DH = D // _LANES

  mesh = plsc.VectorSubcoreMesh(core_axis_name="c", subcore_axis_name="s",
                                num_cores=_NC, num_subcores=_NS)

  def body(ids_hbm, table_hbm, out_hbm, ids_v, buf0, buf1, out_v,
           sem0, sem1):
    bufs = (buf0, buf1)
    sems = (sem0, sem1)
    wid = lax.axis_index("c") * _NS + lax.axis_index("s")

    def fire(g, b):
      descs = []
      for r in range(GROUP):
        for off, sz in SPLITS:
          irow = g * GROUP + r
          dst = bufs[b].at[pl.ds(r * L + off, sz)]
          descs.append(pltpu.async_copy(
              table_hbm.at[ids_v.at[irow, pl.ds(off, sz)]],
              dst, sems[b]))
      return descs

    def reduce_group(g, b):
      buf = bufs[b]
      zero = jnp.zeros((_LANES,), jnp.float32)

      def rbody(j, carry):
        cs = list(carry)
        for u in range(2):
          for r in range(GROUP):
            row = r * L + 2 * j + u
            for h in range(DH):
              k = r * DH + h
              cs[k] = cs[k] + buf[row, pl.ds(h * _LANES, _LANES)]
        return tuple(cs)

      acc = lax.fori_loop(0, L // 2, rbody, (zero,) * (GROUP * DH))
      for r in range(GROUP):
        for h in range(DH):
          out_v[g * GROUP + r, pl.ds(h * _LANES, _LANES)] = acc[r * DH + h]

    def oc_body(oc, carry):
      row_base = wid * BPW + oc * OUTC
      pltpu.sync_copy(ids_hbm.at[pl.ds(row_base, OUTC)], ids_v)
      descs = [None, None]
      descs[0] = fire(0, 0)
      for g in range(NG):
        b = g % 2
        if g + 1 < NG:
          descs[1 - b] = fire(g + 1, 1 - b)
        for dsc in descs[b]:
          dsc.wait()
        reduce_group(g, b)
      pltpu.sync_copy(out_v, out_hbm.at[pl.ds(wid * BPW + oc * OUTC, OUTC)])
      return carry

    lax.fori_loop(0, NOC, oc_body, 0)

  k = pl.kernel(
      body,
      out_type=jax.ShapeDtypeStruct((B, D), jnp.float32),
      mesh=mesh,
      compiler_params=pltpu.CompilerParams(use_tc_tiling_on_sc=False),
      name="sc_embed_pool",
      scratch_types=[
          pltpu.VMEM((OUTC, L), jnp.int32),
          pltpu.VMEM((ROWS, D), jnp.float32),
          pltpu.VMEM((ROWS, D), jnp.float32),
          pltpu.VMEM((OUTC, D), jnp.float32),
          pltpu.SemaphoreType.DMA,
          pltpu.SemaphoreType.DMA,
      ],
  )
  return k(ids, table)


def _mlp(sums, input_ids, W1, b1, W2, b2):
  B, L = input_ids.shape
  D = sums.shape[1]
  H = W1.shape[0]
  T = W2.shape[0]
  BLK = 1024

  def body(sums_ref, ids_ref, W1_ref, b1_ref, W2_ref, b2_ref, out_ref):
    ids = ids_ref[...]
    cnt = jnp.sum((ids != 0).astype(jnp.float32), axis=1, keepdims=True)
    mean = sums_ref[...] / jnp.maximum(cnt, 1.0)
    h = lax.dot_general(mean, W1_ref[...], (((1,), (1,)), ((), ())),
                        preferred_element_type=jnp.float32) + b1_ref[...]
    h = jnp.maximum(h, 0.0)
    out_ref[...] = lax.dot_general(h, W2_ref[...], (((1,), (1,)), ((), ())),
                                   preferred_element_type=jnp.float32) + b2_ref[...]

  return pl.pallas_call(
      body,
      grid=(B // BLK,),
      in_specs=[
          pl.BlockSpec((BLK, D), lambda i: (i, 0)),
          pl.BlockSpec((BLK, L), lambda i: (i, 0)),
          pl.BlockSpec((H, D), lambda i: (0, 0)),
          pl.BlockSpec((1, H), lambda i: (0, 0)),
          pl.BlockSpec((T, H), lambda i: (0, 0)),
          pl.BlockSpec((1, T), lambda i: (0, 0)),
      ],
      out_specs=pl.BlockSpec((BLK, T), lambda i: (i, 0)),
      out_shape=jax.ShapeDtypeStruct((B, T), jnp.float32),
  )(sums, input_ids, W1, b1.reshape(1, H), W2, b2.reshape(1, T))


def kernel(input_ids, table, W1, b1, W2, b2):
  B, L = input_ids.shape
  D = table.shape[1]
  sums = _sc_pool(input_ids, table, B, L, D)
  return _mlp(sums, input_ids, W1, b1, W2, b2)
